# baseline (speedup 1.0000x reference)
import functools

import jax
import jax.numpy as jnp
from jax import lax
from jax.experimental import pallas as pl
from jax.experimental.pallas import tpu as pltpu
from jax.experimental.pallas import tpu_sc as plsc

VOCAB = 1000000
D = 32
B = 4096
L = 200
NC = 2
NS = 16
NW = NC * NS
BPW = B // NW
CH0, CH1 = 104, 96
NBUF = 4

NBLK = VOCAB // 128
TAIL = VOCAB - NBLK * 128
NITER = (NBLK + NW - 1) // NW

_mesh = plsc.VectorSubcoreMesh(core_axis_name="c", subcore_axis_name="s")


@functools.partial(
    pl.kernel,
    mesh=_mesh,
    out_type=jax.ShapeDtypeStruct((VOCAB * D,), jnp.float32),
    scratch_types=[
        [pltpu.VMEM((D, 128), jnp.float32) for _ in range(2)],
        [pltpu.VMEM((128 * D,), jnp.float32) for _ in range(2)],
        pltpu.VMEM((D, TAIL), jnp.float32),
        [pltpu.SemaphoreType.DMA for _ in range(2)],
        [pltpu.SemaphoreType.DMA for _ in range(2)],
    ],
    compiler_params=pltpu.CompilerParams(
        use_tc_tiling_on_sc=True, needs_layout_passes=False),
)
def _transpose_table(tblt_hbm, out_hbm, vbufs, obufs, tbuf, isems, osems):
    wid = lax.axis_index("s") * NC + lax.axis_index("c")

    def fire_in(blk, s):
        col = pl.multiple_of(blk * 128, 128)
        pltpu.async_copy(tblt_hbm.at[:, pl.ds(col, 128)], vbufs[s], isems[s])

    def shuffle(vbuf, obuf, n):
        iota = lax.iota(jnp.int32, 16)

        def group(gg, carry):
            for gs in range(2):
                cols = iota + 32 * gg + 16 * gs
                sbase = cols * D
                for h in range(2):
                    for k in range(16):
                        rot = (iota + k) & 15
                        rows = rot + (16 * h)
                        v = plsc.load_gather(vbuf, [rows, cols])
                        plsc.store_scatter(obuf, [sbase + rows], v)
            return carry

        lax.fori_loop(0, n // 32, group, 0)

    fire_in(wid, 0)

    def body(g, carry):
        for s in range(2):
            i = 2 * g + s
            blk = wid + NW * i

            @pl.when(blk < NBLK)
            def _():
                pltpu.make_async_copy(
                    tblt_hbm.at[:, pl.ds(0, 128)], vbufs[s],
                    isems[s]).wait()

                @pl.when(blk + NW < NBLK)
                def _():
                    fire_in(blk + NW, 1 - s)

                @pl.when(i >= 2)
                def _():
                    pltpu.make_async_copy(
                        obufs[s], out_hbm.at[pl.ds(0, 128 * D)],
                        osems[s]).wait()

                shuffle(vbufs[s], obufs[s], 128)
                dst = pl.multiple_of(blk * (128 * D), 128)
                pltpu.async_copy(
                    obufs[s], out_hbm.at[pl.ds(dst, 128 * D)], osems[s])

        return carry

    lax.fori_loop(0, (NITER + 1) // 2, body, 0)
    for s in range(2):
        pltpu.make_async_copy(
            obufs[s], out_hbm.at[pl.ds(0, 128 * D)], osems[s]).wait()

    @pl.when(wid == lax.rem(NBLK, NW))
    def _():
        pltpu.sync_copy(tblt_hbm.at[:, pl.ds(NBLK * 128, TAIL)], tbuf)
        shuffle(tbuf, obufs[0], TAIL)
        pltpu.sync_copy(obufs[0].at[pl.ds(0, TAIL * D)],
                        out_hbm.at[pl.ds(NBLK * 128 * D, TAIL * D)])


@functools.partial(
    pl.kernel,
    mesh=_mesh,
    out_type=jax.ShapeDtypeStruct((B * D,), jnp.float32),
    scratch_types=[
        pltpu.VMEM((BPW, L), jnp.int32),
        [pltpu.VMEM((L, D), jnp.float32) for _ in range(NBUF)],
        pltpu.VMEM((BPW * D,), jnp.float32),
        [pltpu.SemaphoreType.DMA for _ in range(NBUF)],
    ],
    compiler_params=pltpu.CompilerParams(use_tc_tiling_on_sc=False),
)
def _emb_avg(table_hbm, idx_hbm, out_hbm, idx_v, bufs, out_v, sems):
    wid = lax.axis_index("s") * NC + lax.axis_index("c")
    pltpu.sync_copy(idx_hbm.at[pl.ds(wid * BPW, BPW)], idx_v)

    def fire(row, buf, sem):
        pltpu.async_copy(
            table_hbm.at[idx_v.at[row, pl.ds(0, CH0)]],
            buf.at[pl.ds(0, CH0)], sem)
        pltpu.async_copy(
            table_hbm.at[idx_v.at[row, pl.ds(CH0, CH1)]],
            buf.at[pl.ds(CH0, CH1)], sem)

    def drain(buf, sem):
        pltpu.make_async_copy(
            table_hbm.at[pl.ds(0, CH0)], buf.at[pl.ds(0, CH0)], sem).wait()
        pltpu.make_async_copy(
            table_hbm.at[pl.ds(0, CH1)], buf.at[pl.ds(CH0, CH1)], sem).wait()

    def reduce_store(row, buf):
        accs = [jnp.zeros((16,), jnp.float32) for _ in range(8)]
        for j in range(L):
            k = (j % 4) * 2
            accs[k] = accs[k] + buf[j, 0:16]
            accs[k + 1] = accs[k + 1] + buf[j, 16:32]
        r0 = ((accs[0] + accs[2]) + (accs[4] + accs[6])) * (1.0 / L)
        r1 = ((accs[1] + accs[3]) + (accs[5] + accs[7])) * (1.0 / L)
        out_v[pl.ds(row * D, 16)] = r0
        out_v[pl.ds(row * D + 16, 16)] = r1

    for s in range(NBUF):
        fire(s, bufs[s], sems[s])

    def body(g, carry):
        for s in range(NBUF):
            row = g * NBUF + s
            drain(bufs[s], sems[s])
            reduce_store(row, bufs[s])

            @pl.when(row + NBUF < BPW)
            def _():
                fire(row + NBUF, bufs[s], sems[s])

        return carry

    lax.fori_loop(0, BPW // NBUF, body, 0)
    pltpu.sync_copy(out_v, out_hbm.at[pl.ds(wid * (BPW * D), BPW * D)])


def kernel(embedding_table, input):
    lin = _transpose_table(embedding_table.T)
    out = _emb_avg(lin.reshape(VOCAB, D), input)
    return out.reshape(B, 1, D)


# speedup vs baseline: 1.3208x; 1.3208x over previous
import functools

import jax
import jax.numpy as jnp
from jax import lax
from jax.experimental import pallas as pl
from jax.experimental.pallas import tpu as pltpu
from jax.experimental.pallas import tpu_sc as plsc

VOCAB = 1000000
D = 32
B = 4096
L = 200
NC = 2
NS = 16
NW = NC * NS
BPW = B // NW
CH0, CH1 = 104, 96
NBUF = 4

NBLK = VOCAB // 128
TAIL = VOCAB - NBLK * 128
NITER = (NBLK + NW - 1) // NW

_mesh = plsc.VectorSubcoreMesh(core_axis_name="c", subcore_axis_name="s")


@functools.partial(
    pl.kernel,
    mesh=_mesh,
    out_type=jax.ShapeDtypeStruct((VOCAB * D,), jnp.float32),
    scratch_types=[
        [pltpu.VMEM((D, 128), jnp.float32) for _ in range(2)],
        [pltpu.VMEM((128 * D,), jnp.float32) for _ in range(2)],
        pltpu.VMEM((D, TAIL), jnp.float32),
        [pltpu.SemaphoreType.DMA for _ in range(2)],
        [pltpu.SemaphoreType.DMA for _ in range(2)],
    ],
    compiler_params=pltpu.CompilerParams(
        use_tc_tiling_on_sc=True, needs_layout_passes=False),
)
def _transpose_table(tblt_hbm, out_hbm, vbufs, obufs, tbuf, isems, osems):
    wid = lax.axis_index("s") * NC + lax.axis_index("c")

    def fire_in(blk, s):
        col = pl.multiple_of(blk * 128, 128)
        pltpu.async_copy(tblt_hbm.at[:, pl.ds(col, 128)], vbufs[s], isems[s])

    def shuffle(vbuf, obuf, n):
        iota = lax.iota(jnp.int32, 16)

        def group(gg, carry):
            cols = iota + 16 * gg
            sbase = cols * D
            for h in range(2):
                for k in range(16):
                    rot = (iota + k) & 15
                    rows = rot + (16 * h)
                    v = plsc.load_gather(vbuf, [rows, cols])
                    plsc.store_scatter(obuf, [sbase + rows], v)
            return carry

        lax.fori_loop(0, n // 16, group, 0)

    fire_in(wid, 0)

    def body(g, carry):
        for s in range(2):
            i = 2 * g + s
            blk = wid + NW * i

            @pl.when(blk < NBLK)
            def _():
                pltpu.make_async_copy(
                    tblt_hbm.at[:, pl.ds(0, 128)], vbufs[s],
                    isems[s]).wait()

                @pl.when(blk + NW < NBLK)
                def _():
                    fire_in(blk + NW, 1 - s)

                @pl.when(i >= 2)
                def _():
                    pltpu.make_async_copy(
                        obufs[s], out_hbm.at[pl.ds(0, 128 * D)],
                        osems[s]).wait()

                shuffle(vbufs[s], obufs[s], 128)
                dst = pl.multiple_of(blk * (128 * D), 128)
                pltpu.async_copy(
                    obufs[s], out_hbm.at[pl.ds(dst, 128 * D)], osems[s])

        return carry

    lax.fori_loop(0, (NITER + 1) // 2, body, 0)
    for s in range(2):
        pltpu.make_async_copy(
            obufs[s], out_hbm.at[pl.ds(0, 128 * D)], osems[s]).wait()

    @pl.when(wid == lax.rem(NBLK, NW))
    def _():
        pltpu.sync_copy(tblt_hbm.at[:, pl.ds(NBLK * 128, TAIL)], tbuf)
        shuffle(tbuf, obufs[0], TAIL)
        pltpu.sync_copy(obufs[0].at[pl.ds(0, TAIL * D)],
                        out_hbm.at[pl.ds(NBLK * 128 * D, TAIL * D)])


@functools.partial(
    pl.kernel,
    mesh=_mesh,
    out_type=jax.ShapeDtypeStruct((B * D,), jnp.float32),
    scratch_types=[
        pltpu.VMEM((BPW, L), jnp.int32),
        [pltpu.VMEM((L, D), jnp.float32) for _ in range(NBUF)],
        pltpu.VMEM((BPW * D,), jnp.float32),
        [pltpu.SemaphoreType.DMA for _ in range(NBUF)],
    ],
    compiler_params=pltpu.CompilerParams(use_tc_tiling_on_sc=False),
)
def _emb_avg(table_hbm, idx_hbm, out_hbm, idx_v, bufs, out_v, sems):
    wid = lax.axis_index("s") * NC + lax.axis_index("c")
    pltpu.sync_copy(idx_hbm.at[pl.ds(wid * BPW, BPW)], idx_v)

    def fire(row, buf, sem):
        pltpu.async_copy(
            table_hbm.at[idx_v.at[row, pl.ds(0, CH0)]],
            buf.at[pl.ds(0, CH0)], sem)
        pltpu.async_copy(
            table_hbm.at[idx_v.at[row, pl.ds(CH0, CH1)]],
            buf.at[pl.ds(CH0, CH1)], sem)

    def drain(buf, sem):
        pltpu.make_async_copy(
            table_hbm.at[pl.ds(0, CH0)], buf.at[pl.ds(0, CH0)], sem).wait()
        pltpu.make_async_copy(
            table_hbm.at[pl.ds(0, CH1)], buf.at[pl.ds(CH0, CH1)], sem).wait()

    def reduce_store(row, buf):
        accs = [jnp.zeros((16,), jnp.float32) for _ in range(8)]
        for j in range(L):
            k = (j % 4) * 2
            accs[k] = accs[k] + buf[j, 0:16]
            accs[k + 1] = accs[k + 1] + buf[j, 16:32]
        r0 = ((accs[0] + accs[2]) + (accs[4] + accs[6])) * (1.0 / L)
        r1 = ((accs[1] + accs[3]) + (accs[5] + accs[7])) * (1.0 / L)
        out_v[pl.ds(row * D, 16)] = r0
        out_v[pl.ds(row * D + 16, 16)] = r1

    for s in range(NBUF):
        fire(s, bufs[s], sems[s])

    def body(g, carry):
        for s in range(NBUF):
            row = g * NBUF + s
            drain(bufs[s], sems[s])
            reduce_store(row, bufs[s])

            @pl.when(row + NBUF < BPW)
            def _():
                fire(row + NBUF, bufs[s], sems[s])

        return carry

    lax.fori_loop(0, BPW // NBUF, body, 0)
    pltpu.sync_copy(out_v, out_hbm.at[pl.ds(wid * (BPW * D), BPW * D)])


def kernel(embedding_table, input):
    lin = _transpose_table(embedding_table.T)
    out = _emb_avg(lin.reshape(VOCAB, D), input)
    return out.reshape(B, 1, D)
